# Initial kernel scaffold; baseline (speedup 1.0000x reference)
import jax
import jax.numpy as jnp
from jax.experimental import pallas as pl


def kernel(x, edge_index, edge_weight, batch, W1, b1, W2, b2, W3, b3, gamma, beta, Wl, bl):
    raise NotImplementedError("write your pallas kernel here")


# speedup vs baseline: 7.1498x; 7.1498x over previous
import functools

import jax
import jax.numpy as jnp
from jax import lax
from jax.experimental import pallas as pl
from jax.experimental.pallas import tpu as pltpu
from jax.experimental.pallas import tpu_sc as plsc

N = 10000
E = 320000
H = 128
G = 64
NC = 2
NS = 16
NW = NC * NS
NP = 10240
CK = 128
CHUNKS = 80
EPAD = NW * CHUNKS * CK
PAD_NODE = N + 128

f32 = jnp.float32
i32 = jnp.int32

_ZB = 2048


def _dot(a, b):
    return lax.dot_general(a, b, (((1,), (0,)), ((), ())),
                           precision=lax.Precision.HIGHEST,
                           preferred_element_type=f32)


def _sc_scalar_body(src_hbm, dst_hbm, w_hbm, x_hbm, batch_hbm,
                    a0_out, sa_out,
                    xv, bv, sbuf, dbuf, wbuf, msgbuf, saibuf, zbuf,
                    acc_a0, acc_sa):
    c = lax.axis_index("c")
    s = lax.axis_index("s")
    wid = s * NC + c

    zero16 = jnp.zeros((16,), f32)

    def _z(i, carry):
        zbuf[pl.ds(i * 16, 16)] = zero16
        return carry
    lax.fori_loop(0, _ZB // 16, _z, None)

    a0_seg = NP // NS
    pltpu.sync_copy(zbuf.at[pl.ds(0, a0_seg)],
                    acc_a0.at[pl.ds(s * a0_seg, a0_seg)])
    sa_seg = (G * NP) // NS
    for k in range(sa_seg // _ZB):
        pltpu.sync_copy(zbuf, acc_sa.at[pl.ds(s * sa_seg + k * _ZB, _ZB)])

    pltpu.sync_copy(x_hbm, xv)
    pltpu.sync_copy(batch_hbm, bv)
    plsc.subcore_barrier()

    def body(it, carry):
        base = wid * (CHUNKS * CK) + it * CK
        pltpu.sync_copy(src_hbm.at[pl.ds(base, CK)], sbuf)
        pltpu.sync_copy(dst_hbm.at[pl.ds(base, CK)], dbuf)
        pltpu.sync_copy(w_hbm.at[pl.ds(base, CK)], wbuf)
        for j in range(CK // 16):
            sv = sbuf[pl.ds(j * 16, 16)]
            dv = dbuf[pl.ds(j * 16, 16)]
            wv = wbuf[pl.ds(j * 16, 16)]
            xg = plsc.load_gather(xv, [sv])
            msgbuf[pl.ds(j * 16, 16)] = wv * xg
            gb = plsc.load_gather(bv, [dv])
            saibuf[pl.ds(j * 16, 16)] = gb * NP + sv
        pltpu.sync_copy(msgbuf, acc_a0.at[dbuf], add=True)
        pltpu.sync_copy(wbuf, acc_sa.at[saibuf], add=True)
        return carry
    lax.fori_loop(0, CHUNKS, body, None)

    plsc.subcore_barrier()
    pltpu.sync_copy(acc_a0.at[pl.ds(s * a0_seg, a0_seg)],
                    a0_out.at[c, pl.ds(s * a0_seg, a0_seg)])
    pltpu.sync_copy(acc_sa.at[pl.ds(s * sa_seg, sa_seg)],
                    sa_out.at[c, pl.ds(s * sa_seg, sa_seg)])


_sc_scalar = pl.kernel(
    _sc_scalar_body,
    out_type=[jax.ShapeDtypeStruct((NC, NP), f32),
              jax.ShapeDtypeStruct((NC, G * NP), f32)],
    mesh=plsc.VectorSubcoreMesh(core_axis_name="c", subcore_axis_name="s"),
    compiler_params=pltpu.CompilerParams(needs_layout_passes=False),
    scratch_types=[
        pltpu.VMEM((NP,), f32),
        pltpu.VMEM((NP,), i32),
        pltpu.VMEM((CK,), i32),
        pltpu.VMEM((CK,), i32),
        pltpu.VMEM((CK,), f32),
        pltpu.VMEM((CK,), f32),
        pltpu.VMEM((CK,), i32),
        pltpu.VMEM((_ZB,), f32),
        pltpu.VMEM_SHARED((NP,), f32),
        pltpu.VMEM_SHARED((G * NP,), f32),
    ],
)


def _sc_vec_body(src_hbm, dst_hbm, w_hbm, h1_hbm,
                 a1_out,
                 sbuf, dbuf, wbuf, rows, zrows, acc, gsem):
    c = lax.axis_index("c")
    s = lax.axis_index("s")
    wid = s * NC + c

    zero16 = jnp.zeros((16,), f32)
    for r in range(16):
        for q in range(H // 16):
            zrows[r, pl.ds(q * 16, 16)] = zero16

    row_seg = NP // NS
    for k in range(row_seg // 16):
        pltpu.sync_copy(zrows, acc.at[pl.ds(s * row_seg + k * 16, 16), :])
    plsc.subcore_barrier()

    def body(it, carry):
        base = wid * (CHUNKS * CK) + it * CK
        pltpu.sync_copy(src_hbm.at[pl.ds(base, CK)], sbuf)
        pltpu.sync_copy(dst_hbm.at[pl.ds(base, CK)], dbuf)
        pltpu.sync_copy(w_hbm.at[pl.ds(base, CK)], wbuf)
        pltpu.async_copy(h1_hbm.at[sbuf], rows, gsem).wait()

        def mul(e, carry2):
            wsp = plsc.load_gather(wbuf, [jnp.full((16,), e, i32)])
            for q in range(H // 16):
                rows[e, pl.ds(q * 16, 16)] = rows[e, pl.ds(q * 16, 16)] * wsp
            return carry2
        lax.fori_loop(0, CK, mul, None)

        pltpu.sync_copy(rows, acc.at[dbuf], add=True)
        return carry
    lax.fori_loop(0, CHUNKS, body, None)

    plsc.subcore_barrier()
    pltpu.sync_copy(acc.at[pl.ds(s * row_seg, row_seg), :],
                    a1_out.at[c, pl.ds(s * row_seg, row_seg), :])


_sc_vec = pl.kernel(
    _sc_vec_body,
    out_type=[jax.ShapeDtypeStruct((NC, NP, H), f32)],
    mesh=plsc.VectorSubcoreMesh(core_axis_name="c", subcore_axis_name="s"),
    compiler_params=pltpu.CompilerParams(needs_layout_passes=False),
    scratch_types=[
        pltpu.VMEM((CK,), i32),
        pltpu.VMEM((CK,), i32),
        pltpu.VMEM((CK,), f32),
        pltpu.VMEM((CK, H), f32),
        pltpu.VMEM((16, H), f32),
        pltpu.VMEM_SHARED((NP, H), f32),
        pltpu.SemaphoreType.DMA,
    ],
)


_BLK1 = 1280


def _tc1_body(a0p_ref, w1_ref, b1_ref, a0s_ref, h1_ref):
    p = a0p_ref[...]
    a0 = (p[0] + p[1])[:, None]
    a0s_ref[...] = a0
    h1_ref[...] = jnp.maximum(a0 * w1_ref[...] + b1_ref[...], 0.0)


def _tc1(a0p, w1, b1r):
    return pl.pallas_call(
        _tc1_body,
        grid=(NP // _BLK1,),
        in_specs=[
            pl.BlockSpec((2, _BLK1), lambda i: (0, i)),
            pl.BlockSpec((1, H), lambda i: (0, 0)),
            pl.BlockSpec((1, H), lambda i: (0, 0)),
        ],
        out_specs=[
            pl.BlockSpec((_BLK1, 1), lambda i: (i, 0)),
            pl.BlockSpec((_BLK1, H), lambda i: (i, 0)),
        ],
        out_shape=[jax.ShapeDtypeStruct((NP, 1), f32),
                   jax.ShapeDtypeStruct((NP, H), f32)],
    )(a0p, w1, b1r)


_BLK2 = 1280


def _tc2_body(a1p_ref, a0s_ref, w2a_ref, w2b_ref, b2_ref, h2_ref):
    a1 = a1p_ref[0] + a1p_ref[1]
    acc = _dot(a1, w2a_ref[...])
    h2_ref[...] = jnp.maximum(acc + a0s_ref[...] * w2b_ref[...] + b2_ref[...],
                              0.0)


def _tc2(a1p, a0s, w2a, w2b, b2r):
    return pl.pallas_call(
        _tc2_body,
        grid=(NP // _BLK2,),
        in_specs=[
            pl.BlockSpec((2, _BLK2, H), lambda i: (0, i, 0)),
            pl.BlockSpec((_BLK2, 1), lambda i: (i, 0)),
            pl.BlockSpec((H, H + 1), lambda i: (0, 0)),
            pl.BlockSpec((1, H + 1), lambda i: (0, 0)),
            pl.BlockSpec((1, H + 1), lambda i: (0, 0)),
        ],
        out_specs=pl.BlockSpec((_BLK2, H + 1), lambda i: (i, 0)),
        out_shape=jax.ShapeDtypeStruct((NP, H + 1), f32),
    )(a1p, a0s, w2a, w2b, b2r)


_BLK3 = 1280


def _tc3_body(sa_ref, h2_ref, h1_ref, bt_ref, w3a_ref, w3b_ref, b3_ref,
              gamma_ref, beta_ref, wl_ref, bl_ref, out_ref, acc1, acc2):
    i = pl.program_id(0)

    @pl.when(i == 0)
    def _():
        acc1[...] = jnp.zeros_like(acc1)
        acc2[...] = jnp.zeros_like(acc2)

    sa = sa_ref[0] + sa_ref[1]
    acc1[...] += _dot(sa, h2_ref[...])
    acc2[...] += _dot(sa, h1_ref[...])

    @pl.when(i == pl.num_programs(0) - 1)
    def _():
        bt = bt_ref[...]
        gi = lax.broadcasted_iota(i32, (G, NP // 128, 128), 0)
        eq = (bt[None] == gi).astype(f32)
        cnts = jnp.sum(eq, axis=(1, 2))
        sums = (_dot(acc1[...], w3a_ref[...]) + _dot(acc2[...], w3b_ref[...])
                + cnts[:, None] * b3_ref[...])
        gp = sums / jnp.maximum(cnts, 1.0)[:, None]
        mean = jnp.mean(gp, axis=0, keepdims=True)
        var = jnp.mean((gp - mean) ** 2, axis=0, keepdims=True)
        gn = (gp - mean) * lax.rsqrt(var + 1e-5) * gamma_ref[...] + beta_ref[...]
        out_ref[...] = _dot(gn, wl_ref[...]) + bl_ref[...]


def _tc3(sa3, h2, h1, bt, w3a, w3b, b3r, gammar, betar, wl, blr):
    d = 2 * H + 1
    return pl.pallas_call(
        _tc3_body,
        grid=(NP // _BLK3,),
        in_specs=[
            pl.BlockSpec((2, G, _BLK3), lambda i: (0, 0, i)),
            pl.BlockSpec((_BLK3, H + 1), lambda i: (i, 0)),
            pl.BlockSpec((_BLK3, H), lambda i: (i, 0)),
            pl.BlockSpec((NP // 128, 128), lambda i: (0, 0)),
            pl.BlockSpec((H + 1, d), lambda i: (0, 0)),
            pl.BlockSpec((H, d), lambda i: (0, 0)),
            pl.BlockSpec((1, d), lambda i: (0, 0)),
            pl.BlockSpec((1, d), lambda i: (0, 0)),
            pl.BlockSpec((1, d), lambda i: (0, 0)),
            pl.BlockSpec((d, 1), lambda i: (0, 0)),
            pl.BlockSpec((1, 1), lambda i: (0, 0)),
        ],
        out_specs=pl.BlockSpec((G, 1), lambda i: (0, 0)),
        out_shape=jax.ShapeDtypeStruct((G, 1), f32),
        scratch_shapes=[pltpu.VMEM((G, H + 1), f32),
                        pltpu.VMEM((G, H), f32)],
    )(sa3, h2, h1, bt, w3a, w3b, b3r, gammar, betar, wl, blr)


def kernel(x, edge_index, edge_weight, batch, W1, b1, W2, b2, W3, b3,
           gamma, beta, Wl, bl):
    pad_e = EPAD - E
    src = jnp.concatenate(
        [edge_index[0].astype(i32), jnp.full((pad_e,), PAD_NODE, i32)])
    dst = jnp.concatenate(
        [edge_index[1].astype(i32), jnp.full((pad_e,), PAD_NODE, i32)])
    wgt = jnp.concatenate([edge_weight.astype(f32), jnp.zeros((pad_e,), f32)])
    xp = jnp.pad(x[:, 0].astype(f32), (0, NP - N))
    batch_sc = jnp.pad(batch.astype(i32), (0, NP - N))
    batch_tc = jnp.pad(batch.astype(i32), (0, NP - N),
                       constant_values=-1).reshape(NP // 128, 128)

    a0p, sap = _sc_scalar(src, dst, wgt, xp, batch_sc)
    a0s, h1 = _tc1(a0p, W1.astype(f32), b1.reshape(1, H).astype(f32))
    (a1p,) = _sc_vec(src, dst, wgt, h1)
    h2 = _tc2(a1p, a0s, W2[:H].astype(f32), W2[H:H + 1].astype(f32),
              b2.reshape(1, H + 1).astype(f32))
    d = 2 * H + 1
    out = _tc3(sap.reshape(NC, G, NP), h2, h1, batch_tc,
               W3[:H + 1].astype(f32), W3[H + 1:].astype(f32),
               b3.reshape(1, d).astype(f32), gamma.reshape(1, d).astype(f32),
               beta.reshape(1, d).astype(f32), Wl.astype(f32),
               bl.reshape(1, 1).astype(f32))
    return out
